# baseline (speedup 1.0000x reference)
import jax
import jax.numpy as jnp
from jax import lax
from jax.experimental import pallas as pl
from jax.experimental.pallas import tpu as pltpu
from jax.experimental.pallas import tpu_sc as plsc

N_POST = 100000
N_PRE = 50000
N_SYN = 3200000

NC = 2
NS = 16
N_TILES = NC * NS
BLK = 128
N_BLOCKS = N_SYN // BLK
TILE_BLOCKS = N_BLOCKS // N_TILES
EXTRA_BLOCKS = N_BLOCKS - TILE_BLOCKS * N_TILES
CB = 32
N_MAIN = TILE_BLOCKS // CB
TAIL_B = TILE_BLOCKS - N_MAIN * CB
ACC = 102400
SLICE = ACC // NS
UNROLL = 4


def _sc_body(inp_hbm, idx_hbm, w_hbm, out_hbm,
             inp_v, idx0, idx1, w0, w1, post0, post1, val0, val1,
             idx_t, w_t, post_t, val_t, acc,
             sem_in, semd0, semd1, sem0, sem1, sem_t):
    c = lax.axis_index("c")
    s = lax.axis_index("s")
    wid = c * NS + s
    start_block = wid * TILE_BLOCKS

    in_cp = pltpu.async_copy(inp_hbm, inp_v, sem_in)

    idxs = (idx0, idx1)
    ws = (w0, w1)
    posts = (post0, post1)
    vals = (val0, val1)
    semds = (semd0, semd1)
    sems = (sem0, sem1)
    scatters = [None, None]

    def _fire_dma(k):
        b = k & 1
        blk = start_block + k * CB
        return (
            pltpu.async_copy(idx_hbm.at[pl.ds(blk * 2 * BLK, CB * 2 * BLK)],
                             idxs[b], semds[b]),
            pltpu.async_copy(w_hbm.at[pl.ds(blk * BLK, CB * BLK)], ws[b],
                             semds[b]),
        )

    dmas = [_fire_dma(0), None]

    def _zero(i, _):
        val1[pl.ds(i * 16, 16)] = jnp.zeros((16,), jnp.float32)
        return _
    lax.fori_loop(0, SLICE // 2 // 16, _zero, None)
    pltpu.sync_copy(val1.at[pl.ds(0, SLICE // 2)],
                    acc.at[pl.ds(s * SLICE, SLICE // 2)])
    pltpu.sync_copy(val1.at[pl.ds(0, SLICE // 2)],
                    acc.at[pl.ds(s * SLICE + SLICE // 2, SLICE // 2)])
    in_cp.wait()

    plsc.subcore_barrier()

    def _compute(idx_v, w_v, post_v, val_v, nblocks):
        @plsc.parallel_loop(0, nblocks)
        def _block(j):
            ib = j * 256
            lb = j * 128
            for t in range(8):
                post16 = idx_v[pl.ds(ib + t * 16, 16)]
                pre16 = idx_v[pl.ds(ib + 128 + t * 16, 16)]
                inp16 = plsc.load_gather(inp_v, [pre16])
                post_v[pl.ds(lb + t * 16, 16)] = jnp.where(
                    inp16 > 0.0, post16, jnp.full((16,), -1, jnp.int32))
                val_v[pl.ds(lb + t * 16, 16)] = w_v[pl.ds(lb + t * 16, 16)]

    for k in range(N_MAIN):
        b = k & 1
        for d in dmas[b]:
            d.wait()
        if k + 1 < N_MAIN:
            if scatters[1 - b] is not None:
                scatters[1 - b].wait()
            dmas[1 - b] = _fire_dma(k + 1)
        _compute(idxs[b], ws[b], posts[b], vals[b], CB)
        scatters[b] = pltpu.async_copy(
            vals[b], acc.at[plsc.Indices(posts[b], ignored_value=-1)],
            sems[b], add=True)

    tb = start_block + N_MAIN * CB
    pltpu.sync_copy(idx_hbm.at[pl.ds(tb * 2 * BLK, TAIL_B * 2 * BLK)], idx_t)
    pltpu.sync_copy(w_hbm.at[pl.ds(tb * BLK, TAIL_B * BLK)], w_t)
    _compute(idx_t, w_t, post_t, val_t, TAIL_B)
    tail_cp = pltpu.async_copy(
        val_t, acc.at[plsc.Indices(post_t, ignored_value=-1)],
        sem_t, add=True)

    scatters[0].wait()
    scatters[1].wait()

    @pl.when(wid == 0)
    def _extra():
        eb = N_TILES * TILE_BLOCKS
        pltpu.sync_copy(
            idx_hbm.at[pl.ds(eb * 2 * BLK, EXTRA_BLOCKS * 2 * BLK)],
            idx0.at[pl.ds(0, EXTRA_BLOCKS * 2 * BLK)])
        pltpu.sync_copy(w_hbm.at[pl.ds(eb * BLK, EXTRA_BLOCKS * BLK)],
                        w0.at[pl.ds(0, EXTRA_BLOCKS * BLK)])
        _compute(idx0, w0, post0, val0, EXTRA_BLOCKS)

        def _mask_rest(i, _):
            post0[pl.ds(EXTRA_BLOCKS * BLK + i * 16, 16)] = jnp.full(
                (16,), -1, jnp.int32)
            return _
        lax.fori_loop(0, (CB - EXTRA_BLOCKS) * BLK // 16, _mask_rest, None)
        pltpu.sync_copy(
            val0, acc.at[plsc.Indices(post0, ignored_value=-1)], add=True)

    tail_cp.wait()

    plsc.subcore_barrier()

    pltpu.sync_copy(acc.at[pl.ds(s * SLICE, SLICE)],
                    out_hbm.at[pl.ds(c * ACC + s * SLICE, SLICE)])


def _tc_add_body(p_ref, o_ref):
    o_ref[...] = (p_ref[0, :, :N_POST] + p_ref[1, :, :N_POST])


def kernel(inputs_t, indices, weights):
    inp = inputs_t.reshape(N_PRE)
    idx_flat = (
        indices.reshape(N_BLOCKS, BLK, 2).transpose(0, 2, 1).reshape(-1)
    )
    mesh = plsc.VectorSubcoreMesh(core_axis_name="c", subcore_axis_name="s")
    sc = pl.kernel(
        _sc_body,
        out_type=jax.ShapeDtypeStruct((NC * ACC,), jnp.float32),
        mesh=mesh,
        compiler_params=pltpu.CompilerParams(
            use_tc_tiling_on_sc=False, needs_layout_passes=False
        ),
        scratch_types=[
            pltpu.VMEM((N_PRE,), jnp.float32),
            pltpu.VMEM((CB * 2 * BLK,), jnp.int32),
            pltpu.VMEM((CB * 2 * BLK,), jnp.int32),
            pltpu.VMEM((CB * BLK,), jnp.float32),
            pltpu.VMEM((CB * BLK,), jnp.float32),
            pltpu.VMEM((CB * BLK,), jnp.int32),
            pltpu.VMEM((CB * BLK,), jnp.int32),
            pltpu.VMEM((CB * BLK,), jnp.float32),
            pltpu.VMEM((CB * BLK,), jnp.float32),
            pltpu.VMEM((TAIL_B * 2 * BLK,), jnp.int32),
            pltpu.VMEM((TAIL_B * BLK,), jnp.float32),
            pltpu.VMEM((TAIL_B * BLK,), jnp.int32),
            pltpu.VMEM((TAIL_B * BLK,), jnp.float32),
            pltpu.VMEM_SHARED((ACC,), jnp.float32),
            pltpu.SemaphoreType.DMA,
            pltpu.SemaphoreType.DMA,
            pltpu.SemaphoreType.DMA,
            pltpu.SemaphoreType.DMA,
            pltpu.SemaphoreType.DMA,
            pltpu.SemaphoreType.DMA,
        ],
    )
    partial = sc(inp, idx_flat, weights)
    out = pl.pallas_call(
        _tc_add_body,
        out_shape=jax.ShapeDtypeStruct((1, N_POST), jnp.float32),
    )(partial.reshape(NC, 1, ACC))
    return out


# speedup vs baseline: 1.0415x; 1.0415x over previous
import jax
import jax.numpy as jnp
from jax import lax
from jax.experimental import pallas as pl
from jax.experimental.pallas import tpu as pltpu
from jax.experimental.pallas import tpu_sc as plsc

N_POST = 100000
N_PRE = 50000
N_SYN = 3200000

NC = 2
NS = 16
N_TILES = NC * NS
BLK = 128
N_BLOCKS = N_SYN // BLK
TILE_BLOCKS = N_BLOCKS // N_TILES
EXTRA_BLOCKS = N_BLOCKS - TILE_BLOCKS * N_TILES
CB = 32
N_MAIN = TILE_BLOCKS // CB
TAIL_B = TILE_BLOCKS - N_MAIN * CB
ACC = 102400
SLICE = ACC // NS
UNROLL = 4


def _sc_body(inp_hbm, idx_hbm, w_hbm, out_hbm,
             inp_v, idx0, idx1, w0, w1, post0, post1, zb,
             idx_t, w_t, post_t, acc,
             sem_in, semd0, semd1, sem0, sem1, sem_t):
    c = lax.axis_index("c")
    s = lax.axis_index("s")
    wid = c * NS + s
    start_block = wid * TILE_BLOCKS

    in_cp = pltpu.async_copy(inp_hbm, inp_v, sem_in)

    idxs = (idx0, idx1)
    ws = (w0, w1)
    posts = (post0, post1)
    semds = (semd0, semd1)
    sems = (sem0, sem1)
    scatters = [None, None]

    def _fire_dma(k):
        b = k & 1
        blk = start_block + k * CB
        return (
            pltpu.async_copy(idx_hbm.at[pl.ds(blk * 2 * BLK, CB * 2 * BLK)],
                             idxs[b], semds[b]),
            pltpu.async_copy(w_hbm.at[pl.ds(blk * BLK, CB * BLK)], ws[b],
                             semds[b]),
        )

    dmas = [_fire_dma(0), None]

    def _zero(i, _):
        zb[pl.ds(i * 16, 16)] = jnp.zeros((16,), jnp.float32)
        return _
    lax.fori_loop(0, SLICE // 2 // 16, _zero, None)
    pltpu.sync_copy(zb, acc.at[pl.ds(s * SLICE, SLICE // 2)])
    pltpu.sync_copy(zb, acc.at[pl.ds(s * SLICE + SLICE // 2, SLICE // 2)])
    in_cp.wait()

    plsc.subcore_barrier()

    def _compute(idx_v, post_v, nblocks):
        @plsc.parallel_loop(0, nblocks)
        def _block(j):
            ib = j * 256
            lb = j * 128
            for t in range(8):
                post16 = idx_v[pl.ds(ib + t * 16, 16)]
                pre16 = idx_v[pl.ds(ib + 128 + t * 16, 16)]
                inp16 = plsc.load_gather(inp_v, [pre16])
                post_v[pl.ds(lb + t * 16, 16)] = jnp.where(
                    inp16 > 0.0, post16, jnp.full((16,), -1, jnp.int32))

    for k in range(N_MAIN):
        b = k & 1
        for d in dmas[b]:
            d.wait()
        if k + 1 < N_MAIN:
            if scatters[1 - b] is not None:
                scatters[1 - b].wait()
            dmas[1 - b] = _fire_dma(k + 1)
        _compute(idxs[b], posts[b], CB)
        scatters[b] = pltpu.async_copy(
            ws[b], acc.at[plsc.Indices(posts[b], ignored_value=-1)],
            sems[b], add=True)

    tb = start_block + N_MAIN * CB
    pltpu.sync_copy(idx_hbm.at[pl.ds(tb * 2 * BLK, TAIL_B * 2 * BLK)], idx_t)
    pltpu.sync_copy(w_hbm.at[pl.ds(tb * BLK, TAIL_B * BLK)], w_t)
    _compute(idx_t, post_t, TAIL_B)
    tail_cp = pltpu.async_copy(
        w_t, acc.at[plsc.Indices(post_t, ignored_value=-1)],
        sem_t, add=True)

    scatters[0].wait()
    scatters[1].wait()

    @pl.when(wid == 0)
    def _extra():
        eb = N_TILES * TILE_BLOCKS
        pltpu.sync_copy(
            idx_hbm.at[pl.ds(eb * 2 * BLK, EXTRA_BLOCKS * 2 * BLK)],
            idx0.at[pl.ds(0, EXTRA_BLOCKS * 2 * BLK)])
        pltpu.sync_copy(w_hbm.at[pl.ds(eb * BLK, EXTRA_BLOCKS * BLK)],
                        w0.at[pl.ds(0, EXTRA_BLOCKS * BLK)])
        _compute(idx0, post0, EXTRA_BLOCKS)

        def _mask_rest(i, _):
            post0[pl.ds(EXTRA_BLOCKS * BLK + i * 16, 16)] = jnp.full(
                (16,), -1, jnp.int32)
            return _
        lax.fori_loop(0, (CB - EXTRA_BLOCKS) * BLK // 16, _mask_rest, None)
        pltpu.sync_copy(
            w0, acc.at[plsc.Indices(post0, ignored_value=-1)], add=True)

    tail_cp.wait()

    plsc.subcore_barrier()

    pltpu.sync_copy(acc.at[pl.ds(s * SLICE, SLICE)],
                    out_hbm.at[pl.ds(c * ACC + s * SLICE, SLICE)])


def _tc_add_body(p_ref, o_ref):
    o_ref[...] = (p_ref[0, :, :N_POST] + p_ref[1, :, :N_POST])


def kernel(inputs_t, indices, weights):
    inp = inputs_t.reshape(N_PRE)
    idx_flat = (
        indices.reshape(N_BLOCKS, BLK, 2).transpose(0, 2, 1).reshape(-1)
    )
    mesh = plsc.VectorSubcoreMesh(core_axis_name="c", subcore_axis_name="s")
    sc = pl.kernel(
        _sc_body,
        out_type=jax.ShapeDtypeStruct((NC * ACC,), jnp.float32),
        mesh=mesh,
        compiler_params=pltpu.CompilerParams(
            use_tc_tiling_on_sc=False, needs_layout_passes=False
        ),
        scratch_types=[
            pltpu.VMEM((N_PRE,), jnp.float32),
            pltpu.VMEM((CB * 2 * BLK,), jnp.int32),
            pltpu.VMEM((CB * 2 * BLK,), jnp.int32),
            pltpu.VMEM((CB * BLK,), jnp.float32),
            pltpu.VMEM((CB * BLK,), jnp.float32),
            pltpu.VMEM((CB * BLK,), jnp.int32),
            pltpu.VMEM((CB * BLK,), jnp.int32),
            pltpu.VMEM((SLICE // 2,), jnp.float32),
            pltpu.VMEM((TAIL_B * 2 * BLK,), jnp.int32),
            pltpu.VMEM((TAIL_B * BLK,), jnp.float32),
            pltpu.VMEM((TAIL_B * BLK,), jnp.int32),
            pltpu.VMEM_SHARED((ACC,), jnp.float32),
            pltpu.SemaphoreType.DMA,
            pltpu.SemaphoreType.DMA,
            pltpu.SemaphoreType.DMA,
            pltpu.SemaphoreType.DMA,
            pltpu.SemaphoreType.DMA,
            pltpu.SemaphoreType.DMA,
        ],
    )
    partial = sc(inp, idx_flat, weights)
    out = pl.pallas_call(
        _tc_add_body,
        out_shape=jax.ShapeDtypeStruct((1, N_POST), jnp.float32),
    )(partial.reshape(NC, 1, ACC))
    return out
